# baseline (speedup 1.0000x reference)
import functools

import jax
import jax.numpy as jnp
import numpy as np
from jax.experimental import pallas as pl
from jax.sharding import Mesh, PartitionSpec as P

_N = 16384
_K = 1024
_D = 128
_BN = 4096


def _kmeans_gap_kernel(x_ref, c_ref, o_ref):
    x = x_ref[...]
    c = c_ref[...]
    c2h = 0.5 * jnp.sum(c * c, axis=1, keepdims=True)
    cx = jax.lax.dot_general(
        c, x, (((1,), (1,)), ((), ())),
        preferred_element_type=jnp.float32)
    st = c2h - cx

    p1 = jnp.minimum(st[:512], st[512:])
    p2 = jnp.maximum(st[:512], st[512:])
    for w in (256, 128, 64, 32, 16, 8, 4, 2, 1):
        a1, b1 = p1[:w], p1[w:]
        a2, b2 = p2[:w], p2[w:]
        p1 = jnp.minimum(a1, b1)
        p2 = jnp.minimum(jnp.maximum(a1, b1), jnp.minimum(a2, b2))
    o_ref[...] = 2.0 * (p2[0] - p1[0])


def _gap_impl(x, centroids):
    n = x.shape[0]
    bn = min(_BN, n)
    return pl.pallas_call(
        _kmeans_gap_kernel,
        grid=(n // bn,),
        in_specs=[
            pl.BlockSpec((bn, _D), lambda i: (i, 0)),
            pl.BlockSpec((_K, _D), lambda i: (0, 0)),
        ],
        out_specs=pl.BlockSpec((bn,), lambda i: (i,)),
        out_shape=jax.ShapeDtypeStruct((n,), jnp.float32),
    )(x, centroids)


@jax.jit
def kernel(x, centroids):
    devs = jax.devices()
    nd = 2 if len(devs) >= 2 and x.shape[0] % (2 * _BN) == 0 else 1
    if nd == 1:
        return _gap_impl(x, centroids)
    mesh = Mesh(np.asarray(devs[:nd]), ("q",))
    f = jax.shard_map(
        _gap_impl, mesh=mesh,
        in_specs=(P("q", None), P(None, None)),
        out_specs=P("q"), check_vma=False)
    return f(x, centroids)


# speedup vs baseline: 23.8497x; 23.8497x over previous
import functools

import jax
import jax.numpy as jnp
from jax.experimental import pallas as pl

_N = 16384
_K = 1024
_D = 128
_BN = 4096


def _kmeans_gap_kernel(x_ref, c_ref, o_ref):
    x = x_ref[...]
    c = c_ref[...]
    c2h = 0.5 * jnp.sum(c * c, axis=1, keepdims=True)
    cx = jax.lax.dot_general(
        c, x, (((1,), (1,)), ((), ())),
        preferred_element_type=jnp.float32)
    st = c2h - cx

    p1 = jnp.minimum(st[:512], st[512:])
    p2 = jnp.maximum(st[:512], st[512:])
    for w in (256, 128, 64, 32, 16, 8, 4, 2, 1):
        a1, b1 = p1[:w], p1[w:]
        a2, b2 = p2[:w], p2[w:]
        p1 = jnp.minimum(a1, b1)
        p2 = jnp.minimum(jnp.maximum(a1, b1), jnp.minimum(a2, b2))
    o_ref[...] = 2.0 * (p2[0] - p1[0])


@jax.jit
def kernel(x, centroids):
    grid = (_N // _BN,)
    return pl.pallas_call(
        _kmeans_gap_kernel,
        grid=grid,
        in_specs=[
            pl.BlockSpec((_BN, _D), lambda i: (i, 0)),
            pl.BlockSpec((_K, _D), lambda i: (0, 0)),
        ],
        out_specs=pl.BlockSpec((_BN,), lambda i: (i,)),
        out_shape=jax.ShapeDtypeStruct((_N,), jnp.float32),
    )(x, centroids)


# speedup vs baseline: 24.0351x; 1.0078x over previous
import jax
import jax.numpy as jnp
from jax.experimental import pallas as pl

_N = 16384
_K = 1024
_D = 128
_BN = 4096


def _kmeans_gap_kernel(x_ref, c_ref, o_ref):
    x = x_ref[...]
    c = c_ref[...]
    c2h = 0.5 * jnp.sum(c * c, axis=1, keepdims=True)
    cx = jax.lax.dot_general(
        c, x, (((1,), (1,)), ((), ())),
        preferred_element_type=jnp.float32)
    st = c2h - cx

    p1 = jnp.minimum(st[:512], st[512:])
    p2 = jnp.maximum(st[:512], st[512:])
    for w in (256, 128, 64, 32, 16, 8, 4, 2, 1):
        a1, b1 = p1[:w], p1[w:]
        a2, b2 = p2[:w], p2[w:]
        p1 = jnp.minimum(a1, b1)
        p2 = jnp.minimum(jnp.maximum(a1, b1), jnp.minimum(a2, b2))
    o_ref[...] = 2.0 * (p2[0] - p1[0])


@jax.jit
def kernel(x, centroids):
    grid = (_N // _BN,)
    return pl.pallas_call(
        _kmeans_gap_kernel,
        grid=grid,
        in_specs=[
            pl.BlockSpec((_BN, _D), lambda i: (i, 0)),
            pl.BlockSpec((_K, _D), lambda i: (0, 0)),
        ],
        out_specs=pl.BlockSpec((_BN,), lambda i: (i,)),
        out_shape=jax.ShapeDtypeStruct((_N,), jnp.float32),
    )(x, centroids)
